# Initial kernel scaffold; baseline (speedup 1.0000x reference)
import functools

import jax
import jax.numpy as jnp
from jax import lax
from jax.experimental import pallas as pl
from jax.experimental.pallas import tpu as pltpu
from jax.experimental.pallas import tpu_sc as plsc

NN = 10000
HID = 128
NC = 2
NS = 16
NW = NC * NS
LANES = 16
EK = 128

_mesh = lambda: plsc.VectorSubcoreMesh(
    core_axis_name="c", subcore_axis_name="s", num_cores=NC, num_subcores=NS)


def _zero_rows(buf, nrows):
  z = jnp.zeros((LANES,), jnp.float32)

  def body(i, _):
    for j in range(HID // LANES):
      buf[i, pl.ds(j * LANES, LANES)] = z
    return 0

  lax.fori_loop(0, nrows, body, 0)


def _deg_body(nchunks, dst_hbm, w_hbm, out_hbm, idx_v, w_v, stage, deg_sh):
  c = lax.axis_index("c")
  s = lax.axis_index("s")
  wid = s * NC + c

  z = jnp.zeros((LANES,), jnp.float32)

  def zb(i, _):
    stage[pl.ds(i * LANES, LANES)] = z
    return 0

  lax.fori_loop(0, 640 // LANES, zb, 0)

  @pl.when(s < NS - 1)
  def _():
    pltpu.sync_copy(stage, deg_sh.at[pl.ds(s * 640, 640)])

  @pl.when(s == NS - 1)
  def _():
    pltpu.sync_copy(stage.at[pl.ds(0, 400)], deg_sh.at[pl.ds(9600, 400)])

  plsc.subcore_barrier()

  nloc = (nchunks - wid + NW - 1) // NW

  def chunk(j, _):
    base = (wid + j * NW) * EK
    pltpu.sync_copy(dst_hbm.at[pl.ds(base, EK)], idx_v)
    pltpu.sync_copy(w_hbm.at[pl.ds(base, EK)], w_v)
    pltpu.sync_copy(w_v, deg_sh.at[idx_v], add=True)
    return 0

  lax.fori_loop(0, nloc, chunk, 0)
  plsc.subcore_barrier()

  @pl.when(s < NS - 1)
  def _():
    pltpu.sync_copy(deg_sh.at[pl.ds(s * 640, 640)], stage)
    pltpu.sync_copy(stage, out_hbm.at[pl.ds(c * NN + s * 640, 640)])

  @pl.when(s == NS - 1)
  def _():
    pltpu.sync_copy(deg_sh.at[pl.ds(9600, 400)], stage.at[pl.ds(0, 400)])
    pltpu.sync_copy(stage.at[pl.ds(0, 400)],
                    out_hbm.at[pl.ds(c * NN + 9600, 400)])


def _deg_call(dst, w):
  e = dst.shape[0]
  nchunks = e // EK
  body = functools.partial(_deg_body, nchunks)
  return pl.kernel(
      body,
      out_type=jax.ShapeDtypeStruct((NC * NN,), jnp.float32),
      mesh=_mesh(),
      scratch_types=[
          pltpu.VMEM((EK,), jnp.int32),
          pltpu.VMEM((EK,), jnp.float32),
          pltpu.VMEM((640,), jnp.float32),
          pltpu.VMEM_SHARED((NN,), jnp.float32),
      ],
  )(dst, w)


def _agg_body(nchunks, h_hbm, src_hbm, dst_hbm, w_hbm, out_hbm,
              ids_v, idd_v, w_v, rows_v, acc_sh, sem):
  c = lax.axis_index("c")
  s = lax.axis_index("s")
  wid = s * NC + c

  _zero_rows(rows_v, EK)
  for k in range(5):
    pltpu.sync_copy(rows_v.at[pl.ds(0, 125)],
                    acc_sh.at[pl.ds(s * 625 + k * 125, 125)])
  plsc.subcore_barrier()

  nloc = (nchunks - wid + NW - 1) // NW

  def chunk(j, _):
    base = (wid + j * NW) * EK
    pltpu.sync_copy(src_hbm.at[pl.ds(base, EK)], ids_v)
    pltpu.sync_copy(dst_hbm.at[pl.ds(base, EK)], idd_v)
    pltpu.sync_copy(w_hbm.at[pl.ds(base, EK)], w_v)
    pltpu.async_copy(h_hbm.at[ids_v], rows_v, sem).wait()

    def row(r, _):
      wv = plsc.load_gather(w_v, [jnp.full((LANES,), r, jnp.int32)])
      for j8 in range(HID // LANES):
        sl = pl.ds(j8 * LANES, LANES)
        rows_v[r, sl] = rows_v[r, sl] * wv
      return 0

    lax.fori_loop(0, EK, row, 0)
    pltpu.sync_copy(rows_v, acc_sh.at[idd_v], add=True)
    return 0

  lax.fori_loop(0, nloc, chunk, 0)
  plsc.subcore_barrier()

  for k in range(5):
    r0 = s * 625 + k * 125
    pltpu.sync_copy(acc_sh.at[pl.ds(r0, 125)], rows_v.at[pl.ds(0, 125)])
    pltpu.sync_copy(rows_v.at[pl.ds(0, 125)],
                    out_hbm.at[pl.ds(c * NN + r0, 125)])


def _agg_call(h, src, dst, w):
  e = src.shape[0]
  nchunks = e // EK
  body = functools.partial(_agg_body, nchunks)
  return pl.kernel(
      body,
      out_type=jax.ShapeDtypeStruct((NC * NN, HID), jnp.float32),
      mesh=_mesh(),
      scratch_types=[
          pltpu.VMEM((EK,), jnp.int32),
          pltpu.VMEM((EK,), jnp.int32),
          pltpu.VMEM((EK,), jnp.float32),
          pltpu.VMEM((EK, HID), jnp.float32),
          pltpu.VMEM_SHARED((NN, HID), jnp.float32),
          pltpu.SemaphoreType.DMA,
      ],
  )(h, src, dst, w)


def _pre_body(x_ref, w1_ref, degp_ref, h1_ref, dis_ref):
  deg = jnp.sum(degp_ref[...], axis=1, keepdims=True) + 1.0
  dis = jnp.where(deg > 0, lax.rsqrt(jnp.maximum(deg, 1e-12)), 0.0)
  h = lax.dot_general(x_ref[...], w1_ref[...], (((1,), (1,)), ((), ())),
                      preferred_element_type=jnp.float32)
  h1_ref[...] = dis * h
  dis_ref[...] = dis


def _pre_call(x, w1, degp_t):
  return pl.pallas_call(
      _pre_body,
      out_shape=(
          jax.ShapeDtypeStruct((NN, HID), jnp.float32),
          jax.ShapeDtypeStruct((NN, 1), jnp.float32),
      ),
  )(x, w1, degp_t)


def _bn(a, g, be):
  m = jnp.mean(a, axis=0, keepdims=True)
  d = a - m
  v = jnp.mean(d * d, axis=0, keepdims=True)
  return d * lax.rsqrt(v + 1e-5) * g + be


def _post_body(p0_ref, p1_ref, hp_ref, dis_ref, b_ref, g_ref, be_ref, w_ref,
               out_ref):
  agg = p0_ref[...] + p1_ref[...] + hp_ref[...]
  pre = dis_ref[...] * agg + b_ref[...]
  a = jnp.maximum(pre, 0.0)
  xn = _bn(a, g_ref[...], be_ref[...])
  h = lax.dot_general(xn, w_ref[...], (((1,), (1,)), ((), ())),
                      preferred_element_type=jnp.float32)
  out_ref[...] = dis_ref[...] * h


def _post_call(p0, p1, hp, dis, b, g, be, w):
  return pl.pallas_call(
      _post_body,
      out_shape=jax.ShapeDtypeStruct((NN, HID), jnp.float32),
  )(p0, p1, hp, dis, b, g, be, w)


def _final_body(p0_ref, p1_ref, hp_ref, dis_ref, b_ref, g_ref, be_ref,
                wl_ref, bl_ref, logits_ref, soft_ref, x3_ref):
  agg = p0_ref[...] + p1_ref[...] + hp_ref[...]
  pre = dis_ref[...] * agg + b_ref[...]
  x3 = _bn(pre, g_ref[...], be_ref[...])
  logits = lax.dot_general(x3, wl_ref[...], (((1,), (1,)), ((), ())),
                           preferred_element_type=jnp.float32) + bl_ref[...]
  mx = jnp.max(logits, axis=1, keepdims=True)
  ex = jnp.exp(logits - mx)
  soft = ex / jnp.sum(ex, axis=1, keepdims=True)
  logits_ref[...] = logits
  soft_ref[...] = soft
  x3_ref[...] = x3


def _final_call(p0, p1, hp, dis, b, g, be, wl, bl):
  nc = wl.shape[0]
  return pl.pallas_call(
      _final_body,
      out_shape=(
          jax.ShapeDtypeStruct((NN, nc), jnp.float32),
          jax.ShapeDtypeStruct((NN, nc), jnp.float32),
          jax.ShapeDtypeStruct((NN, HID), jnp.float32),
      ),
  )(p0, p1, hp, dis, b, g, be, wl, bl)


def kernel(x, edge_index, edge_weight, W1, b1, g1, be1, W2, b2, g2, be2,
           W3, b3, g3, be3, Wl, bl):
  src = edge_index[0]
  dst = edge_index[1]

  degp = _deg_call(dst, edge_weight)
  degp_t = degp.reshape(NC, NN).T

  h1p, dis = _pre_call(x, W1, degp_t)

  p = _agg_call(h1p, src, dst, edge_weight)
  h2p = _post_call(p[:NN], p[NN:], h1p, dis, b1.reshape(1, HID),
                   g1.reshape(1, HID), be1.reshape(1, HID), W2)

  p = _agg_call(h2p, src, dst, edge_weight)
  h3p = _post_call(p[:NN], p[NN:], h2p, dis, b2.reshape(1, HID),
                   g2.reshape(1, HID), be2.reshape(1, HID), W3)

  p = _agg_call(h3p, src, dst, edge_weight)
  logits, soft, x3 = _final_call(p[:NN], p[NN:], h3p, dis,
                                 b3.reshape(1, HID), g3.reshape(1, HID),
                                 be3.reshape(1, HID), Wl, bl.reshape(1, -1))
  return (logits, soft, x3)


# speedup vs baseline: 10.2315x; 10.2315x over previous
import functools

import jax
import jax.numpy as jnp
from jax import lax
from jax.experimental import pallas as pl
from jax.experimental.pallas import tpu as pltpu
from jax.experimental.pallas import tpu_sc as plsc

NN = 10000
HID = 128
NC = 2
NS = 16
NW = NC * NS
LANES = 16
EK = 128

_mesh = lambda: plsc.VectorSubcoreMesh(
    core_axis_name="c", subcore_axis_name="s", num_cores=NC, num_subcores=NS)


def _zero_rows(buf, nrows):
  z = jnp.zeros((LANES,), jnp.float32)

  def body(i, _):
    for j in range(HID // LANES):
      buf[i, pl.ds(j * LANES, LANES)] = z
    return 0

  lax.fori_loop(0, nrows, body, 0)


def _deg_body(nchunks, dst_hbm, w_hbm, out_hbm, idx_v, w_v, stage, deg_sh):
  c = lax.axis_index("c")
  s = lax.axis_index("s")
  wid = s * NC + c

  z = jnp.zeros((LANES,), jnp.float32)

  def zb(i, _):
    stage[pl.ds(i * LANES, LANES)] = z
    return 0

  lax.fori_loop(0, 640 // LANES, zb, 0)

  @pl.when(s < NS - 1)
  def _():
    pltpu.sync_copy(stage, deg_sh.at[pl.ds(s * 640, 640)])

  @pl.when(s == NS - 1)
  def _():
    pltpu.sync_copy(stage.at[pl.ds(0, 400)], deg_sh.at[pl.ds(9600, 400)])

  plsc.subcore_barrier()

  nloc = (nchunks - wid + NW - 1) // NW

  def chunk(j, _):
    base = (wid + j * NW) * EK
    pltpu.sync_copy(dst_hbm.at[pl.ds(base, EK)], idx_v)
    pltpu.sync_copy(w_hbm.at[pl.ds(base, EK)], w_v)
    pltpu.sync_copy(w_v, deg_sh.at[idx_v], add=True)
    return 0

  lax.fori_loop(0, nloc, chunk, 0)
  plsc.subcore_barrier()

  @pl.when(s < NS - 1)
  def _():
    pltpu.sync_copy(deg_sh.at[pl.ds(s * 640, 640)], stage)
    pltpu.sync_copy(stage, out_hbm.at[pl.ds(c * NN + s * 640, 640)])

  @pl.when(s == NS - 1)
  def _():
    pltpu.sync_copy(deg_sh.at[pl.ds(9600, 400)], stage.at[pl.ds(0, 400)])
    pltpu.sync_copy(stage.at[pl.ds(0, 400)],
                    out_hbm.at[pl.ds(c * NN + 9600, 400)])


def _deg_call(dst, w):
  e = dst.shape[0]
  nchunks = e // EK
  body = functools.partial(_deg_body, nchunks)
  return pl.kernel(
      body,
      out_type=jax.ShapeDtypeStruct((NC * NN,), jnp.float32),
      mesh=_mesh(),
      scratch_types=[
          pltpu.VMEM((EK,), jnp.int32),
          pltpu.VMEM((EK,), jnp.float32),
          pltpu.VMEM((640,), jnp.float32),
          pltpu.VMEM_SHARED((NN,), jnp.float32),
      ],
  )(dst, w)


ROWS_T = 632
ROWS_LAST = NN - ROWS_T * (NS - 1)


def _chunk_sizes(total):
  out = []
  while total > 0:
    c = min(total, EK)
    out.append(c)
    total -= c
  return out


def _agg_body(nchunks, h_hbm, src_hbm, dst_hbm, w_hbm, out_hbm,
              ids_v, idd_v, w_v, rows_v, acc_sh, sem):
  c = lax.axis_index("c")
  s = lax.axis_index("s")
  wid = s * NC + c

  def _tile_rows(fn):
    @pl.when(s < NS - 1)
    def _():
      fn(s * ROWS_T, _chunk_sizes(ROWS_T))

    @pl.when(s == NS - 1)
    def _():
      fn((NS - 1) * ROWS_T, _chunk_sizes(ROWS_LAST))

  _zero_rows(rows_v, EK)

  def _zero_acc(row0, sizes):
    off = 0
    for sz in sizes:
      pltpu.sync_copy(rows_v.at[pl.ds(0, sz)],
                      acc_sh.at[pl.ds(row0 + off, sz)])
      off += sz

  _tile_rows(_zero_acc)
  plsc.subcore_barrier()

  nloc = (nchunks - wid + NW - 1) // NW

  def chunk(j, _):
    base = (wid + j * NW) * EK
    pltpu.sync_copy(src_hbm.at[pl.ds(base, EK)], ids_v)
    pltpu.sync_copy(dst_hbm.at[pl.ds(base, EK)], idd_v)
    pltpu.sync_copy(w_hbm.at[pl.ds(base, EK)], w_v)
    pltpu.async_copy(h_hbm.at[ids_v], rows_v, sem).wait()

    def row16(q, _):
      w16 = w_v[pl.ds(q * LANES, LANES)]
      for i in range(LANES):
        r = q * LANES + i
        wv = jnp.full((LANES,), w16[i], jnp.float32)
        for j8 in range(HID // LANES):
          sl = pl.ds(j8 * LANES, LANES)
          rows_v[r, sl] = rows_v[r, sl] * wv
      return 0

    lax.fori_loop(0, EK // LANES, row16, 0)
    pltpu.sync_copy(rows_v, acc_sh.at[idd_v], add=True)
    return 0

  lax.fori_loop(0, nloc, chunk, 0)
  plsc.subcore_barrier()

  def _copy_out(row0, sizes):
    off = 0
    for sz in sizes:
      pltpu.sync_copy(acc_sh.at[pl.ds(row0 + off, sz)],
                      rows_v.at[pl.ds(0, sz)])
      pltpu.sync_copy(rows_v.at[pl.ds(0, sz)],
                      out_hbm.at[pl.ds(c * NN + row0 + off, sz)])
      off += sz

  _tile_rows(_copy_out)


def _agg_call(h, src, dst, w):
  e = src.shape[0]
  nchunks = e // EK
  body = functools.partial(_agg_body, nchunks)
  return pl.kernel(
      body,
      out_type=jax.ShapeDtypeStruct((NC * NN, HID), jnp.float32),
      mesh=_mesh(),
      scratch_types=[
          pltpu.VMEM((EK,), jnp.int32),
          pltpu.VMEM((EK,), jnp.int32),
          pltpu.VMEM((EK,), jnp.float32),
          pltpu.VMEM((EK, HID), jnp.float32),
          pltpu.VMEM_SHARED((NN, HID), jnp.float32),
          pltpu.SemaphoreType.DMA,
      ],
  )(h, src, dst, w)


def _pre_body(x_ref, w1_ref, degp_ref, h1_ref, dis_ref):
  deg = jnp.sum(degp_ref[...], axis=1, keepdims=True) + 1.0
  dis = jnp.where(deg > 0, lax.rsqrt(jnp.maximum(deg, 1e-12)), 0.0)
  h = lax.dot_general(x_ref[...], w1_ref[...], (((1,), (1,)), ((), ())),
                      preferred_element_type=jnp.float32)
  h1_ref[...] = dis * h
  dis_ref[...] = dis


def _pre_call(x, w1, degp_t):
  return pl.pallas_call(
      _pre_body,
      out_shape=(
          jax.ShapeDtypeStruct((NN, HID), jnp.float32),
          jax.ShapeDtypeStruct((NN, 1), jnp.float32),
      ),
  )(x, w1, degp_t)


def _bn(a, g, be):
  m = jnp.mean(a, axis=0, keepdims=True)
  d = a - m
  v = jnp.mean(d * d, axis=0, keepdims=True)
  return d * lax.rsqrt(v + 1e-5) * g + be


def _post_body(p0_ref, p1_ref, hp_ref, dis_ref, b_ref, g_ref, be_ref, w_ref,
               out_ref):
  agg = p0_ref[...] + p1_ref[...] + hp_ref[...]
  pre = dis_ref[...] * agg + b_ref[...]
  a = jnp.maximum(pre, 0.0)
  xn = _bn(a, g_ref[...], be_ref[...])
  h = lax.dot_general(xn, w_ref[...], (((1,), (1,)), ((), ())),
                      preferred_element_type=jnp.float32)
  out_ref[...] = dis_ref[...] * h


def _post_call(p0, p1, hp, dis, b, g, be, w):
  return pl.pallas_call(
      _post_body,
      out_shape=jax.ShapeDtypeStruct((NN, HID), jnp.float32),
  )(p0, p1, hp, dis, b, g, be, w)


def _final_body(p0_ref, p1_ref, hp_ref, dis_ref, b_ref, g_ref, be_ref,
                wl_ref, bl_ref, logits_ref, soft_ref, x3_ref):
  agg = p0_ref[...] + p1_ref[...] + hp_ref[...]
  pre = dis_ref[...] * agg + b_ref[...]
  x3 = _bn(pre, g_ref[...], be_ref[...])
  logits = lax.dot_general(x3, wl_ref[...], (((1,), (1,)), ((), ())),
                           preferred_element_type=jnp.float32) + bl_ref[...]
  mx = jnp.max(logits, axis=1, keepdims=True)
  ex = jnp.exp(logits - mx)
  soft = ex / jnp.sum(ex, axis=1, keepdims=True)
  logits_ref[...] = logits
  soft_ref[...] = soft
  x3_ref[...] = x3


def _final_call(p0, p1, hp, dis, b, g, be, wl, bl):
  nc = wl.shape[0]
  return pl.pallas_call(
      _final_body,
      out_shape=(
          jax.ShapeDtypeStruct((NN, nc), jnp.float32),
          jax.ShapeDtypeStruct((NN, nc), jnp.float32),
          jax.ShapeDtypeStruct((NN, HID), jnp.float32),
      ),
  )(p0, p1, hp, dis, b, g, be, wl, bl)


def kernel(x, edge_index, edge_weight, W1, b1, g1, be1, W2, b2, g2, be2,
           W3, b3, g3, be3, Wl, bl):
  src = edge_index[0]
  dst = edge_index[1]

  degp = _deg_call(dst, edge_weight)
  degp_t = degp.reshape(NC, NN).T

  h1p, dis = _pre_call(x, W1, degp_t)

  p = _agg_call(h1p, src, dst, edge_weight)
  h2p = _post_call(p[:NN], p[NN:], h1p, dis, b1.reshape(1, HID),
                   g1.reshape(1, HID), be1.reshape(1, HID), W2)

  p = _agg_call(h2p, src, dst, edge_weight)
  h3p = _post_call(p[:NN], p[NN:], h2p, dis, b2.reshape(1, HID),
                   g2.reshape(1, HID), be2.reshape(1, HID), W3)

  p = _agg_call(h3p, src, dst, edge_weight)
  logits, soft, x3 = _final_call(p[:NN], p[NN:], h3p, dis,
                                 b3.reshape(1, HID), g3.reshape(1, HID),
                                 be3.reshape(1, HID), Wl, bl.reshape(1, -1))
  return (logits, soft, x3)
